# baseline (speedup 1.0000x reference)
import functools

import jax
import jax.numpy as jnp
from jax import lax
from jax.experimental import pallas as pl
from jax.experimental.pallas import tpu as pltpu

_CAPS = [0.25, 0.2, 0.15, 0.1, 0.1, 0.08, 0.07, 0.05]
_H = 16


def _router_kernel(x_ref, wr_ref, br_ref, eid_ref, r_ref, *, caps):
    b, n, d = x_ref.shape
    e_tot = wr_ref.shape[1]
    x2 = x_ref[...].reshape(b * n, d)
    logits = jnp.dot(x2, wr_ref[...], preferred_element_type=jnp.float32)
    logits = logits + br_ref[...]
    mx = jnp.max(logits, axis=-1, keepdims=True)
    ex = jnp.exp(logits - mx)
    probs = ex / jnp.sum(ex, axis=-1, keepdims=True)
    pt = probs.T.reshape(e_tot, b, n)

    iota = lax.broadcasted_iota(jnp.int32, (b, n), 1)
    assigned = jnp.zeros((b, n), jnp.bool_)
    eid = jnp.full((b, n), e_tot - 1, jnp.int32)

    for e in range(e_tot - 1):
        cap = caps[e]
        p_e = pt[e]
        bits = lax.bitcast_convert_type(p_e, jnp.int32)
        masked = jnp.where(assigned, jnp.int32(-1), bits)

        def _bs(_, carry, masked=masked, cap=cap):
            lo, hi = carry
            mid = (lo + hi) // 2
            cnt = jnp.sum((masked >= mid).astype(jnp.int32), axis=1,
                          keepdims=True)
            ge = cnt >= cap
            return jnp.where(ge, mid, lo), jnp.where(ge, hi, mid)

        lo0 = jnp.zeros((b, 1), jnp.int32)
        hi0 = jnp.full((b, 1), jnp.int32(0x40000000))
        t, _ = lax.fori_loop(0, 31, _bs, (lo0, hi0))

        gt = masked > t
        eq = masked == t
        n_gt = jnp.sum(gt.astype(jnp.int32), axis=1, keepdims=True)
        need = cap - n_gt

        def _bs2(_, carry, eq=eq, need=need):
            lo2, hi2 = carry
            mid = (lo2 + hi2) // 2
            c = jnp.sum((eq & (iota < mid)).astype(jnp.int32), axis=1,
                        keepdims=True)
            ok = c >= need
            return jnp.where(ok, lo2, mid + 1), jnp.where(ok, mid, hi2)

        nbits = max(1, (n + 1).bit_length())
        _, p = lax.fori_loop(0, nbits, _bs2,
                             (jnp.zeros((b, 1), jnp.int32),
                              jnp.full((b, 1), n, jnp.int32)))
        sel = gt | (eq & (iota < p))
        eid = jnp.where(sel, e, eid)
        assigned = assigned | sel

    r = jnp.zeros((b, n), jnp.float32)
    for e in range(e_tot):
        r = jnp.where(eid == e, pt[e], r)
    eid_ref[...] = eid.reshape(b, n, 1)
    r_ref[...] = r.reshape(b, n, 1)


def _qkv_kernel(x_ref, eid_ref, g_ref, bln_ref, wq_ref, bq_ref, wk_ref,
                bk_ref, wv_ref, bv_ref, q_ref, k_ref, v_ref):
    x = x_ref[0]
    d = x.shape[-1]
    mu = jnp.mean(x, axis=-1, keepdims=True)
    var = jnp.mean((x - mu) ** 2, axis=-1, keepdims=True)
    xn = (x - mu) / jnp.sqrt(var + 1e-5) * g_ref[...] + bln_ref[...]
    m = jnp.int32(d) >> eid_ref[0]
    fm = (lax.broadcasted_iota(jnp.int32, (1, d), 1) < m).astype(x.dtype)
    xm = (xn * fm).astype(jnp.bfloat16)
    q_ref[0] = ((jnp.dot(xm, wq_ref[...],
                         preferred_element_type=jnp.float32) + bq_ref[...])
                * fm).astype(jnp.bfloat16)
    k_ref[0] = ((jnp.dot(xm, wk_ref[...],
                         preferred_element_type=jnp.float32) + bk_ref[...])
                * fm).astype(jnp.bfloat16)
    v_ref[0] = ((jnp.dot(xm, wv_ref[...],
                         preferred_element_type=jnp.float32) + bv_ref[...])
                * fm).astype(jnp.bfloat16)


def _attn_kernel(q_ref, k_ref, v_ref, o_ref, *, dh, scale):
    n = q_ref.shape[1]
    nh = q_ref.shape[-1] // dh
    for i in range(nh):
        sl = slice(i * dh, (i + 1) * dh)
        qh = q_ref[0, :, sl]
        kh = k_ref[0, :, sl]
        vh = v_ref[0, :, sl]
        logits = (lax.dot_general(qh, kh, (((1,), (1,)), ((), ())),
                                  preferred_element_type=jnp.float32)
                  * scale).astype(jnp.bfloat16)
        mx = jnp.max(logits, axis=-1, keepdims=True)
        exb = jnp.exp(logits - mx)
        vext = jnp.concatenate([vh, jnp.ones((n, 1), jnp.bfloat16)], axis=1)
        ovs = jnp.dot(exb, vext, preferred_element_type=jnp.float32)
        rs = 1.0 / ovs[:, dh:dh + 1]
        o_ref[0, :, sl] = (ovs[:, :dh] * rs).astype(jnp.bfloat16)


def _ffn_kernel(x_ref, o_ref, eid_ref, r_ref, wo_ref, bo_ref, g_ref, bln_ref,
                w1_ref, b1_ref, w2_ref, b2_ref, alpha_ref, out_ref):
    x = x_ref[0]
    d = x.shape[-1]
    m = jnp.int32(d) >> eid_ref[0]
    fm = (lax.broadcasted_iota(jnp.int32, (1, d), 1) < m).astype(x.dtype)
    ob = o_ref[0] * fm.astype(jnp.bfloat16)
    op = (jnp.dot(ob, wo_ref[...],
                  preferred_element_type=jnp.float32) + bo_ref[...]) * fm
    z = x + op
    mu = jnp.mean(z, axis=-1, keepdims=True)
    var = jnp.mean((z - mu) ** 2, axis=-1, keepdims=True)
    zn = ((z - mu) / jnp.sqrt(var + 1e-5) * g_ref[...]
          + bln_ref[...]).astype(jnp.bfloat16)
    h = jnp.dot(zn, w1_ref[...],
                preferred_element_type=jnp.float32) + b1_ref[...]
    h = jax.nn.gelu(h.astype(jnp.bfloat16), approximate=True)
    ff = jnp.dot(h, w2_ref[...],
                 preferred_element_type=jnp.float32) + b2_ref[...]
    out_ref[0] = z + r_ref[0] * ff * alpha_ref[...]


@jax.jit
def kernel(x, ln1_g, ln1_b, Wr, br, Wq, bq, Wk, bk, Wv, bv, Wo, bo,
           ln2_g, ln2_b, W1, b1, W2, b2, alpha):
    B, N, D = x.shape
    E = Wr.shape[1]
    D4 = W1.shape[1]
    dh = D // _H
    caps = [int(c * N) for c in _CAPS]

    f32 = jnp.float32
    bf16 = jnp.bfloat16
    row = lambda a: a.reshape(1, -1)
    Wq, Wk, Wv, Wo, W1, W2 = (w.astype(bf16) for w in (Wq, Wk, Wv, Wo, W1, W2))

    eidT, rT = pl.pallas_call(
        functools.partial(_router_kernel, caps=caps),
        out_shape=[jax.ShapeDtypeStruct((B, N, 1), jnp.int32),
                   jax.ShapeDtypeStruct((B, N, 1), f32)],
    )(x, Wr, row(br))

    T = 1024 if N % 1024 == 0 else N
    nt = N // T
    wspec = pl.BlockSpec((D, D), lambda b, t: (0, 0))
    rowspec = pl.BlockSpec((1, D), lambda b, t: (0, 0))
    xspec = pl.BlockSpec((1, T, D), lambda b, t: (b, t, 0))
    espec = pl.BlockSpec((1, T, 1), lambda b, t: (b, t, 0))
    q, k, v = pl.pallas_call(
        _qkv_kernel,
        grid=(B, nt),
        in_specs=[xspec, espec, rowspec, rowspec, wspec, rowspec,
                  wspec, rowspec, wspec, rowspec],
        out_specs=[xspec, xspec, xspec],
        out_shape=[jax.ShapeDtypeStruct((B, N, D), bf16)] * 3,
    )(x, eidT, row(ln1_g), row(ln1_b), Wq, row(bq), Wk, row(bk), Wv, row(bv))

    hpb = max(1, 512 // dh)
    hspec = pl.BlockSpec((1, N, hpb * dh), lambda b, h: (b, 0, h))
    o = pl.pallas_call(
        functools.partial(_attn_kernel, dh=dh, scale=1.0 / (dh ** 0.5)),
        grid=(B, _H // hpb),
        in_specs=[hspec, hspec, hspec],
        out_specs=hspec,
        out_shape=jax.ShapeDtypeStruct((B, N, D), bf16),
    )(q, k, v)

    T2 = 512 if N % 512 == 0 else N
    nt2 = N // T2
    xspec3 = pl.BlockSpec((1, T2, D), lambda b, t: (b, t, 0))
    espec3 = pl.BlockSpec((1, T2, 1), lambda b, t: (b, t, 0))
    out = pl.pallas_call(
        _ffn_kernel,
        grid=(B, nt2),
        in_specs=[xspec3, xspec3, espec3, espec3, wspec, rowspec,
                  rowspec, rowspec,
                  pl.BlockSpec((D, D4), lambda b, t: (0, 0)),
                  pl.BlockSpec((1, D4), lambda b, t: (0, 0)),
                  pl.BlockSpec((D4, D), lambda b, t: (0, 0)),
                  rowspec, rowspec],
        out_specs=xspec3,
        out_shape=jax.ShapeDtypeStruct((B, N, D), f32),
    )(x, o, eidT, rT, Wo, row(bo), row(ln2_g), row(ln2_b),
      W1, row(b1), W2, row(b2), row(alpha))
    return out


# speedup vs baseline: 1.1374x; 1.1374x over previous
import functools

import jax
import jax.numpy as jnp
from jax import lax
from jax.experimental import pallas as pl
from jax.experimental.pallas import tpu as pltpu

_CAPS = [0.25, 0.2, 0.15, 0.1, 0.1, 0.08, 0.07, 0.05]
_H = 16


def _router_kernel(x_ref, wr_ref, br_ref, eid_ref, r_ref, *, caps):
    b, n, d = x_ref.shape
    e_tot = wr_ref.shape[1]
    x2 = x_ref[...].reshape(b * n, d)
    logits = jnp.dot(x2, wr_ref[...], preferred_element_type=jnp.float32)
    logits = logits + br_ref[...]
    mx = jnp.max(logits, axis=-1, keepdims=True)
    ex = jnp.exp(logits - mx)
    probs = ex / jnp.sum(ex, axis=-1, keepdims=True)
    pt = probs.T.reshape(e_tot, b, n)

    iota = lax.broadcasted_iota(jnp.int32, (b, n), 1)
    assigned = jnp.zeros((b, n), jnp.bool_)
    eid = jnp.full((b, n), e_tot - 1, jnp.int32)

    for e in range(e_tot - 1):
        cap = caps[e]
        p_e = pt[e]
        bits = lax.bitcast_convert_type(p_e, jnp.int32)
        masked = jnp.where(assigned, jnp.int32(-1), bits)

        def _bs(_, carry, masked=masked, cap=cap):
            lo, hi = carry
            mid = (lo + hi) // 2
            cnt = jnp.sum((masked >= mid).astype(jnp.int32), axis=1,
                          keepdims=True)
            ge = cnt >= cap
            return jnp.where(ge, mid, lo), jnp.where(ge, hi, mid)

        lo0 = jnp.zeros((b, 1), jnp.int32)
        hi0 = jnp.full((b, 1), jnp.int32(0x40000000))
        t, _ = lax.fori_loop(0, 31, _bs, (lo0, hi0))

        gt = masked > t
        eq = masked == t
        n_gt = jnp.sum(gt.astype(jnp.int32), axis=1, keepdims=True)
        need = cap - n_gt

        def _bs2(_, carry, eq=eq, need=need):
            lo2, hi2 = carry
            mid = (lo2 + hi2) // 2
            c = jnp.sum((eq & (iota < mid)).astype(jnp.int32), axis=1,
                        keepdims=True)
            ok = c >= need
            return jnp.where(ok, lo2, mid + 1), jnp.where(ok, mid, hi2)

        nbits = max(1, (n + 1).bit_length())
        _, p = lax.fori_loop(0, nbits, _bs2,
                             (jnp.zeros((b, 1), jnp.int32),
                              jnp.full((b, 1), n, jnp.int32)))
        sel = gt | (eq & (iota < p))
        eid = jnp.where(sel, e, eid)
        assigned = assigned | sel

    r = jnp.zeros((b, n), jnp.float32)
    for e in range(e_tot):
        r = jnp.where(eid == e, pt[e], r)
    eid_ref[...] = eid.reshape(b, n, 1)
    r_ref[...] = r.reshape(b, n, 1)


def _qkv_kernel(x_ref, eid_ref, g_ref, bln_ref, wq_ref, bq_ref, wk_ref,
                bk_ref, wv_ref, bv_ref, q_ref, k_ref, v_ref):
    x = x_ref[0]
    d = x.shape[-1]
    mu = jnp.mean(x, axis=-1, keepdims=True)
    var = jnp.mean((x - mu) ** 2, axis=-1, keepdims=True)
    xn = (x - mu) / jnp.sqrt(var + 1e-5) * g_ref[...] + bln_ref[...]
    m = jnp.int32(d) >> eid_ref[0]
    fm = (lax.broadcasted_iota(jnp.int32, (1, d), 1) < m).astype(x.dtype)
    xm = (xn * fm).astype(jnp.bfloat16)
    q_ref[0] = ((jnp.dot(xm, wq_ref[...],
                         preferred_element_type=jnp.float32) + bq_ref[...])
                * fm).astype(jnp.bfloat16)
    k_ref[0] = ((jnp.dot(xm, wk_ref[...],
                         preferred_element_type=jnp.float32) + bk_ref[...])
                * fm).astype(jnp.bfloat16)
    v_ref[0] = ((jnp.dot(xm, wv_ref[...],
                         preferred_element_type=jnp.float32) + bv_ref[...])
                * fm).astype(jnp.bfloat16)


def _attn_kernel(q_ref, k_ref, v_ref, o_ref, *, dh, scale):
    n = q_ref.shape[1]
    nh = q_ref.shape[-1] // dh
    for i in range(nh):
        sl = slice(i * dh, (i + 1) * dh)
        qh = q_ref[0, :, sl]
        kh = k_ref[0, :, sl]
        vh = v_ref[0, :, sl]
        logits = (lax.dot_general(qh, kh, (((1,), (1,)), ((), ())),
                                  preferred_element_type=jnp.float32)
                  * scale).astype(jnp.bfloat16)
        mx = jnp.max(logits, axis=-1, keepdims=True)
        exb = jnp.exp(logits - mx)
        vext = jnp.concatenate([vh, jnp.ones((n, 1), jnp.bfloat16)], axis=1)
        ovs = jnp.dot(exb, vext, preferred_element_type=jnp.float32)
        rs = 1.0 / ovs[:, dh:dh + 1]
        o_ref[0, :, sl] = (ovs[:, :dh] * rs).astype(jnp.bfloat16)


def _ffn_kernel(x_ref, o_ref, eid_ref, r_ref, wo_ref, bo_ref, g_ref, bln_ref,
                w1_ref, b1_ref, w2_ref, b2_ref, alpha_ref, out_ref):
    x = x_ref[0]
    d = x.shape[-1]
    m = jnp.int32(d) >> eid_ref[0]
    fm = (lax.broadcasted_iota(jnp.int32, (1, d), 1) < m).astype(x.dtype)
    ob = o_ref[0] * fm.astype(jnp.bfloat16)
    op = (jnp.dot(ob, wo_ref[...],
                  preferred_element_type=jnp.float32) + bo_ref[...]) * fm
    z = x + op
    mu = jnp.mean(z, axis=-1, keepdims=True)
    var = jnp.mean((z - mu) ** 2, axis=-1, keepdims=True)
    zn = ((z - mu) / jnp.sqrt(var + 1e-5) * g_ref[...]
          + bln_ref[...]).astype(jnp.bfloat16)
    h = jnp.dot(zn, w1_ref[...],
                preferred_element_type=jnp.float32) + b1_ref[...]
    h = jax.nn.gelu(h.astype(jnp.bfloat16), approximate=True)
    ff = jnp.dot(h, w2_ref[...],
                 preferred_element_type=jnp.float32) + b2_ref[...]
    out_ref[0] = z + r_ref[0] * ff * alpha_ref[...]


@jax.jit
def kernel(x, ln1_g, ln1_b, Wr, br, Wq, bq, Wk, bk, Wv, bv, Wo, bo,
           ln2_g, ln2_b, W1, b1, W2, b2, alpha):
    B, N, D = x.shape
    E = Wr.shape[1]
    D4 = W1.shape[1]
    dh = D // _H
    caps = [int(c * N) for c in _CAPS]

    f32 = jnp.float32
    bf16 = jnp.bfloat16
    row = lambda a: a.reshape(1, -1)
    Wq, Wk, Wv, Wo, W1, W2 = (w.astype(bf16) for w in (Wq, Wk, Wv, Wo, W1, W2))

    eidT, rT = pl.pallas_call(
        functools.partial(_router_kernel, caps=caps),
        out_shape=[jax.ShapeDtypeStruct((B, N, 1), jnp.int32),
                   jax.ShapeDtypeStruct((B, N, 1), f32)],
    )(x, Wr, row(br))

    T = 1024 if N % 1024 == 0 else N
    nt = N // T
    wspec = pl.BlockSpec((D, D), lambda b, t: (0, 0))
    rowspec = pl.BlockSpec((1, D), lambda b, t: (0, 0))
    xspec = pl.BlockSpec((1, T, D), lambda b, t: (b, t, 0))
    espec = pl.BlockSpec((1, T, 1), lambda b, t: (b, t, 0))
    q, k, v = pl.pallas_call(
        _qkv_kernel,
        grid=(B, nt),
        in_specs=[xspec, espec, rowspec, rowspec, wspec, rowspec,
                  wspec, rowspec, wspec, rowspec],
        out_specs=[xspec, xspec, xspec],
        out_shape=[jax.ShapeDtypeStruct((B, N, D), bf16)] * 3,
    )(x, eidT, row(ln1_g), row(ln1_b), Wq, row(bq), Wk, row(bk), Wv, row(bv))

    hpb = max(1, 256 // dh)
    hspec = pl.BlockSpec((1, N, hpb * dh), lambda b, h: (b, 0, h))
    o = pl.pallas_call(
        functools.partial(_attn_kernel, dh=dh, scale=1.0 / (dh ** 0.5)),
        grid=(B, _H // hpb),
        in_specs=[hspec, hspec, hspec],
        out_specs=hspec,
        out_shape=jax.ShapeDtypeStruct((B, N, D), bf16),
    )(q, k, v)

    T2 = 512 if N % 512 == 0 else N
    nt2 = N // T2
    xspec3 = pl.BlockSpec((1, T2, D), lambda b, t: (b, t, 0))
    espec3 = pl.BlockSpec((1, T2, 1), lambda b, t: (b, t, 0))
    out = pl.pallas_call(
        _ffn_kernel,
        grid=(B, nt2),
        in_specs=[xspec3, xspec3, espec3, espec3, wspec, rowspec,
                  rowspec, rowspec,
                  pl.BlockSpec((D, D4), lambda b, t: (0, 0)),
                  pl.BlockSpec((1, D4), lambda b, t: (0, 0)),
                  pl.BlockSpec((D4, D), lambda b, t: (0, 0)),
                  rowspec, rowspec],
        out_specs=xspec3,
        out_shape=jax.ShapeDtypeStruct((B, N, D), f32),
    )(x, o, eidT, rT, Wo, row(bo), row(ln2_g), row(ln2_b),
      W1, row(b1), W2, row(b2), row(alpha))
    return out


# speedup vs baseline: 1.1446x; 1.0063x over previous
import functools

import jax
import jax.numpy as jnp
from jax import lax
from jax.experimental import pallas as pl
from jax.experimental.pallas import tpu as pltpu

_CAPS = [0.25, 0.2, 0.15, 0.1, 0.1, 0.08, 0.07, 0.05]
_H = 16


def _router_kernel(x_ref, wr_ref, br_ref, eid_ref, r_ref, *, caps):
    b, n, d = x_ref.shape
    e_tot = wr_ref.shape[1]
    x2 = x_ref[...].reshape(b * n, d)
    logits = jnp.dot(x2, wr_ref[...], preferred_element_type=jnp.float32)
    logits = logits + br_ref[...]
    mx = jnp.max(logits, axis=-1, keepdims=True)
    ex = jnp.exp(logits - mx)
    probs = ex / jnp.sum(ex, axis=-1, keepdims=True)
    pt = probs.T.reshape(e_tot, b, n)

    iota = lax.broadcasted_iota(jnp.int32, (b, n), 1)
    assigned = jnp.zeros((b, n), jnp.bool_)
    eid = jnp.full((b, n), e_tot - 1, jnp.int32)

    for e in range(e_tot - 1):
        cap = caps[e]
        p_e = pt[e]
        bits = lax.bitcast_convert_type(p_e, jnp.int32)
        masked = jnp.where(assigned, jnp.int32(-1), bits)

        def _bs(_, carry, masked=masked, cap=cap):
            lo, hi = carry
            mid = (lo + hi) // 2
            cnt = jnp.sum((masked >= mid).astype(jnp.int32), axis=1,
                          keepdims=True)
            ge = cnt >= cap
            return jnp.where(ge, mid, lo), jnp.where(ge, hi, mid)

        lo0 = jnp.zeros((b, 1), jnp.int32)
        hi0 = jnp.full((b, 1), jnp.int32(0x40000000))
        t, _ = lax.fori_loop(0, 31, _bs, (lo0, hi0))

        gt = masked > t
        eq = masked == t
        n_gt = jnp.sum(gt.astype(jnp.int32), axis=1, keepdims=True)
        need = cap - n_gt

        def _bs2(_, carry, eq=eq, need=need):
            lo2, hi2 = carry
            mid = (lo2 + hi2) // 2
            c = jnp.sum((eq & (iota < mid)).astype(jnp.int32), axis=1,
                        keepdims=True)
            ok = c >= need
            return jnp.where(ok, lo2, mid + 1), jnp.where(ok, mid, hi2)

        nbits = max(1, (n + 1).bit_length())
        _, p = lax.fori_loop(0, nbits, _bs2,
                             (jnp.zeros((b, 1), jnp.int32),
                              jnp.full((b, 1), n, jnp.int32)))
        sel = gt | (eq & (iota < p))
        eid = jnp.where(sel, e, eid)
        assigned = assigned | sel

    r = jnp.zeros((b, n), jnp.float32)
    for e in range(e_tot):
        r = jnp.where(eid == e, pt[e], r)
    eid_ref[...] = eid.reshape(b, n, 1)
    r_ref[...] = r.reshape(b, n, 1)


def _qkv_kernel(x_ref, eid_ref, g_ref, bln_ref, wq_ref, bq_ref, wk_ref,
                bk_ref, wv_ref, bv_ref, q_ref, k_ref, v_ref):
    x = x_ref[0]
    d = x.shape[-1]
    mu = jnp.mean(x, axis=-1, keepdims=True)
    var = jnp.mean((x - mu) ** 2, axis=-1, keepdims=True)
    xn = (x - mu) / jnp.sqrt(var + 1e-5) * g_ref[...] + bln_ref[...]
    m = jnp.int32(d) >> eid_ref[0]
    fm = (lax.broadcasted_iota(jnp.int32, (1, d), 1) < m).astype(x.dtype)
    xm = (xn * fm).astype(jnp.bfloat16)
    q_ref[0] = ((jnp.dot(xm, wq_ref[...],
                         preferred_element_type=jnp.float32) + bq_ref[...])
                * fm).astype(jnp.bfloat16)
    k_ref[0] = ((jnp.dot(xm, wk_ref[...],
                         preferred_element_type=jnp.float32) + bk_ref[...])
                * fm).astype(jnp.bfloat16)
    v_ref[0] = ((jnp.dot(xm, wv_ref[...],
                         preferred_element_type=jnp.float32) + bv_ref[...])
                * fm).astype(jnp.bfloat16)


def _attn_kernel(q_ref, k_ref, v_ref, o_ref, *, dh, scale):
    n = k_ref.shape[1]
    nh = q_ref.shape[-1] // dh
    for i in range(nh):
        sl = slice(i * dh, (i + 1) * dh)
        qh = q_ref[0, :, sl]
        kh = k_ref[0, :, sl]
        vh = v_ref[0, :, sl]
        logits = (lax.dot_general(qh, kh, (((1,), (1,)), ((), ())),
                                  preferred_element_type=jnp.float32)
                  * scale).astype(jnp.bfloat16)
        mx = jnp.max(logits, axis=-1, keepdims=True)
        exb = jnp.exp(logits - mx)
        vext = jnp.concatenate([vh, jnp.ones((n, 1), jnp.bfloat16)], axis=1)
        ovs = jnp.dot(exb, vext, preferred_element_type=jnp.float32)
        rs = 1.0 / ovs[:, dh:dh + 1]
        o_ref[0, :, sl] = (ovs[:, :dh] * rs).astype(jnp.bfloat16)


def _ffn_kernel(x_ref, o_ref, eid_ref, r_ref, wo_ref, bo_ref, g_ref, bln_ref,
                w1_ref, b1_ref, w2_ref, b2_ref, alpha_ref, out_ref):
    x = x_ref[0]
    d = x.shape[-1]
    m = jnp.int32(d) >> eid_ref[0]
    fm = (lax.broadcasted_iota(jnp.int32, (1, d), 1) < m).astype(x.dtype)
    ob = o_ref[0] * fm.astype(jnp.bfloat16)
    op = (jnp.dot(ob, wo_ref[...],
                  preferred_element_type=jnp.float32) + bo_ref[...]) * fm
    z = x + op
    mu = jnp.mean(z, axis=-1, keepdims=True)
    var = jnp.mean((z - mu) ** 2, axis=-1, keepdims=True)
    zn = ((z - mu) / jnp.sqrt(var + 1e-5) * g_ref[...]
          + bln_ref[...]).astype(jnp.bfloat16)
    h = jnp.dot(zn, w1_ref[...],
                preferred_element_type=jnp.float32) + b1_ref[...]
    h = jax.nn.gelu(h.astype(jnp.bfloat16), approximate=True)
    ff = jnp.dot(h, w2_ref[...],
                 preferred_element_type=jnp.float32) + b2_ref[...]
    out_ref[0] = z + r_ref[0] * ff * alpha_ref[...]


@jax.jit
def kernel(x, ln1_g, ln1_b, Wr, br, Wq, bq, Wk, bk, Wv, bv, Wo, bo,
           ln2_g, ln2_b, W1, b1, W2, b2, alpha):
    B, N, D = x.shape
    E = Wr.shape[1]
    D4 = W1.shape[1]
    dh = D // _H
    caps = [int(c * N) for c in _CAPS]

    f32 = jnp.float32
    bf16 = jnp.bfloat16
    row = lambda a: a.reshape(1, -1)
    Wq, Wk, Wv, Wo, W1, W2 = (w.astype(bf16) for w in (Wq, Wk, Wv, Wo, W1, W2))

    eidT, rT = pl.pallas_call(
        functools.partial(_router_kernel, caps=caps),
        out_shape=[jax.ShapeDtypeStruct((B, N, 1), jnp.int32),
                   jax.ShapeDtypeStruct((B, N, 1), f32)],
    )(x, Wr, row(br))

    T = 1024 if N % 1024 == 0 else N
    nt = N // T
    wspec = pl.BlockSpec((D, D), lambda b, t: (0, 0))
    rowspec = pl.BlockSpec((1, D), lambda b, t: (0, 0))
    xspec = pl.BlockSpec((1, T, D), lambda b, t: (b, t, 0))
    espec = pl.BlockSpec((1, T, 1), lambda b, t: (b, t, 0))
    q, k, v = pl.pallas_call(
        _qkv_kernel,
        grid=(B, nt),
        in_specs=[xspec, espec, rowspec, rowspec, wspec, rowspec,
                  wspec, rowspec, wspec, rowspec],
        out_specs=[xspec, xspec, xspec],
        out_shape=[jax.ShapeDtypeStruct((B, N, D), bf16)] * 3,
    )(x, eidT, row(ln1_g), row(ln1_b), Wq, row(bq), Wk, row(bk), Wv, row(bv))

    hpb = max(1, 256 // dh)
    NQ = 1024 if N % 1024 == 0 else N
    qspec = pl.BlockSpec((1, NQ, hpb * dh), lambda b, h, t: (b, t, h))
    kvspec = pl.BlockSpec((1, N, hpb * dh), lambda b, h, t: (b, 0, h))
    o = pl.pallas_call(
        functools.partial(_attn_kernel, dh=dh, scale=1.0 / (dh ** 0.5)),
        grid=(B, _H // hpb, N // NQ),
        in_specs=[qspec, kvspec, kvspec],
        out_specs=qspec,
        out_shape=jax.ShapeDtypeStruct((B, N, D), bf16),
    )(q, k, v)

    T2 = 512 if N % 512 == 0 else N
    nt2 = N // T2
    xspec3 = pl.BlockSpec((1, T2, D), lambda b, t: (b, t, 0))
    espec3 = pl.BlockSpec((1, T2, 1), lambda b, t: (b, t, 0))
    out = pl.pallas_call(
        _ffn_kernel,
        grid=(B, nt2),
        in_specs=[xspec3, xspec3, espec3, espec3, wspec, rowspec,
                  rowspec, rowspec,
                  pl.BlockSpec((D, D4), lambda b, t: (0, 0)),
                  pl.BlockSpec((1, D4), lambda b, t: (0, 0)),
                  pl.BlockSpec((D4, D), lambda b, t: (0, 0)),
                  rowspec, rowspec],
        out_specs=xspec3,
        out_shape=jax.ShapeDtypeStruct((B, N, D), f32),
    )(x, o, eidT, rT, Wo, row(bo), row(ln2_g), row(ln2_b),
      W1, row(b1), W2, row(b2), row(alpha))
    return out
